# baseline (speedup 1.0000x reference)
import jax
import jax.numpy as jnp
from jax.experimental import pallas as pl

NUM_EXPERTS = 64
TOP_K = 8
HIDDEN = 768
BLOCK_T = 2048


def _router_block(x_ref, w_ref, b_ref, probs_ref, tw_ref, ti_ref):
    logits = jax.lax.dot_general(
        w_ref[...], x_ref[...],
        dimension_numbers=(((1,), (1,)), ((), ())),
        preferred_element_type=jnp.float32,
    )
    logits = logits + b_ref[...]

    m = jnp.max(logits, axis=0, keepdims=True)
    e = jnp.exp(logits - m)
    s = jnp.sum(e, axis=0, keepdims=True)
    probs_ref[...] = (e / s).T

    iota_f = jax.lax.broadcasted_iota(jnp.int32, e.shape, 0).astype(jnp.float32)
    p = e
    rows_w, rows_i = [], []
    wsum = jnp.zeros((1, e.shape[1]), jnp.float32)
    for _ in range(TOP_K):
        cm = jnp.max(p, axis=0, keepdims=True)
        eq = p == cm
        idx = jnp.min(jnp.where(eq, iota_f, 64.0), axis=0, keepdims=True)
        rows_w.append(cm)
        rows_i.append(idx)
        wsum = wsum + cm
        p = jnp.where(eq, -1.0, p)
    tw_ref[...] = jnp.concatenate(rows_w, axis=0) / wsum
    ti_ref[...] = jnp.concatenate(rows_i, axis=0).astype(jnp.int32)


def kernel(x, gate_weight, expert_bias):
    flat_x = x.reshape(-1, x.shape[-1])
    n_tokens = flat_x.shape[0]
    grid = (n_tokens // BLOCK_T,)
    bias = expert_bias.reshape(NUM_EXPERTS, 1)

    probs, tw_t, ti_t = pl.pallas_call(
        _router_block,
        grid=grid,
        in_specs=[
            pl.BlockSpec((BLOCK_T, HIDDEN), lambda i: (i, 0)),
            pl.BlockSpec((NUM_EXPERTS, HIDDEN), lambda i: (0, 0)),
            pl.BlockSpec((NUM_EXPERTS, 1), lambda i: (0, 0)),
        ],
        out_specs=[
            pl.BlockSpec((BLOCK_T, NUM_EXPERTS), lambda i: (i, 0)),
            pl.BlockSpec((TOP_K, BLOCK_T), lambda i: (0, i)),
            pl.BlockSpec((TOP_K, BLOCK_T), lambda i: (0, i)),
        ],
        out_shape=[
            jax.ShapeDtypeStruct((n_tokens, NUM_EXPERTS), jnp.float32),
            jax.ShapeDtypeStruct((TOP_K, n_tokens), jnp.float32),
            jax.ShapeDtypeStruct((TOP_K, n_tokens), jnp.int32),
        ],
    )(flat_x, gate_weight, bias)
    return (tw_t.T, ti_t.T, probs)


# speedup vs baseline: 1.0636x; 1.0636x over previous
import jax
import jax.numpy as jnp
from jax.experimental import pallas as pl

NUM_EXPERTS = 64
TOP_K = 8
HIDDEN = 768
BLOCK_T = 4096


def _router_block(x_ref, w_ref, b_ref, probs_ref, tw_ref, ti_ref):
    logits = jax.lax.dot_general(
        w_ref[...], x_ref[...],
        dimension_numbers=(((1,), (1,)), ((), ())),
        preferred_element_type=jnp.float32,
    )
    logits = logits + b_ref[...]

    m = jnp.max(logits, axis=0, keepdims=True)
    e = jnp.exp(logits - m)
    s = jnp.sum(e, axis=0, keepdims=True)
    probs_ref[...] = (e / s).T

    iota_f = jax.lax.broadcasted_iota(jnp.int32, e.shape, 0).astype(jnp.float32)
    p = e
    rows_w, rows_i = [], []
    wsum = jnp.zeros((1, e.shape[1]), jnp.float32)
    for _ in range(TOP_K):
        cm = jnp.max(p, axis=0, keepdims=True)
        eq = p == cm
        idx = jnp.min(jnp.where(eq, iota_f, 64.0), axis=0, keepdims=True)
        rows_w.append(cm)
        rows_i.append(idx)
        wsum = wsum + cm
        p = jnp.where(eq, -1.0, p)
    tw_ref[...] = jnp.concatenate(rows_w, axis=0) / wsum
    ti_ref[...] = jnp.concatenate(rows_i, axis=0).astype(jnp.int32)


def kernel(x, gate_weight, expert_bias):
    flat_x = x.reshape(-1, x.shape[-1])
    n_tokens = flat_x.shape[0]
    grid = (n_tokens // BLOCK_T,)
    bias = expert_bias.reshape(NUM_EXPERTS, 1)

    probs, tw_t, ti_t = pl.pallas_call(
        _router_block,
        grid=grid,
        in_specs=[
            pl.BlockSpec((BLOCK_T, HIDDEN), lambda i: (i, 0)),
            pl.BlockSpec((NUM_EXPERTS, HIDDEN), lambda i: (0, 0)),
            pl.BlockSpec((NUM_EXPERTS, 1), lambda i: (0, 0)),
        ],
        out_specs=[
            pl.BlockSpec((BLOCK_T, NUM_EXPERTS), lambda i: (i, 0)),
            pl.BlockSpec((TOP_K, BLOCK_T), lambda i: (0, i)),
            pl.BlockSpec((TOP_K, BLOCK_T), lambda i: (0, i)),
        ],
        out_shape=[
            jax.ShapeDtypeStruct((n_tokens, NUM_EXPERTS), jnp.float32),
            jax.ShapeDtypeStruct((TOP_K, n_tokens), jnp.float32),
            jax.ShapeDtypeStruct((TOP_K, n_tokens), jnp.int32),
        ],
    )(flat_x, gate_weight, bias)
    return (tw_t.T, ti_t.T, probs)
